# baseline (speedup 1.0000x reference)
import jax
import jax.numpy as jnp
from jax import lax
from jax.experimental import pallas as pl
from jax.experimental.pallas import tpu as pltpu

_N = 16384
_S = 1024
_D = 64
_B = 512
_GRID = _N // _B
_GRAPH_BIAS_SCALE = 0.5
_COMMITMENT_COST = 0.25


def _vq_body(z_ref, cb_ref, adj_ref, p_ref,
             zq_ref, idx_ref, loss_ref, pp_ref,
             bias_ref, hist_ref, lacc_ref):
    i = pl.program_id(0)

    @pl.when(i == 0)
    def _init():
        bias_ref[...] = _GRAPH_BIAS_SCALE * jax.nn.sigmoid(adj_ref[...])
        hist_ref[...] = jnp.zeros_like(hist_ref)
        lacc_ref[0, 0] = 0.0

    z = z_ref[...]
    cb = cb_ref[...]
    p = p_ref[...]

    iota_s = lax.broadcasted_iota(jnp.int32, (_B, _S), 1)

    onehot_p = (p == iota_s).astype(jnp.float32)
    bias = lax.dot_general(
        onehot_p, bias_ref[...], (((1,), (0,)), ((), ())),
        precision=lax.Precision.HIGHEST, preferred_element_type=jnp.float32)

    mm = lax.dot_general(
        z, cb, (((1,), (1,)), ((), ())), preferred_element_type=jnp.float32)
    sz = jnp.sum(z * z, axis=-1, keepdims=True)
    ones_row = jnp.ones((1, _D), dtype=jnp.float32)
    sc = lax.dot_general(
        ones_row, cb * cb, (((1,), (1,)), ((), ())),
        precision=lax.Precision.HIGHEST, preferred_element_type=jnp.float32)
    d = (sz + sc) - 2.0 * mm
    d = d - bias

    dmin = jnp.min(d, axis=-1, keepdims=True)
    idx = jnp.min(jnp.where(d == dmin, iota_s, _S), axis=-1, keepdims=True)
    idx_ref[...] = idx

    onehot_q = (idx == iota_s).astype(jnp.float32)
    zq = lax.dot_general(
        onehot_q, cb, (((1,), (0,)), ((), ())),
        precision=lax.Precision.HIGHEST, preferred_element_type=jnp.float32)
    zq_ref[...] = zq
    hist_ref[...] += jnp.sum(onehot_q, axis=0, keepdims=True)
    lacc_ref[0, 0] += jnp.sum((zq - z) ** 2)

    @pl.when(i == _GRID - 1)
    def _fini():
        loss_ref[0, 0] = lacc_ref[0, 0] * ((1.0 + _COMMITMENT_COST) / (_N * _D))
        avg = hist_ref[...] * (1.0 / _N)
        pp_ref[0, 0] = jnp.exp(-jnp.sum(avg * jnp.log(avg + 1e-10)))


def kernel(z_real, z_imag, codebook, adjacency, prev_symbol_idx):
    z_flat = jnp.concatenate([z_real, z_imag], axis=-1)
    p_col = prev_symbol_idx.astype(jnp.int32).reshape(_N, 1)

    zq, idx, loss, pp = pl.pallas_call(
        _vq_body,
        grid=(_GRID,),
        in_specs=[
            pl.BlockSpec((_B, _D), lambda i: (i, 0)),
            pl.BlockSpec((_S, _D), lambda i: (0, 0)),
            pl.BlockSpec((_S, _S), lambda i: (0, 0)),
            pl.BlockSpec((_B, 1), lambda i: (i, 0)),
        ],
        out_specs=[
            pl.BlockSpec((_B, _D), lambda i: (i, 0)),
            pl.BlockSpec((_B, 1), lambda i: (i, 0)),
            pl.BlockSpec(memory_space=pltpu.SMEM, block_shape=(1, 1),
                         index_map=lambda i: (0, 0)),
            pl.BlockSpec(memory_space=pltpu.SMEM, block_shape=(1, 1),
                         index_map=lambda i: (0, 0)),
        ],
        out_shape=[
            jax.ShapeDtypeStruct((_N, _D), jnp.float32),
            jax.ShapeDtypeStruct((_N, 1), jnp.int32),
            jax.ShapeDtypeStruct((1, 1), jnp.float32),
            jax.ShapeDtypeStruct((1, 1), jnp.float32),
        ],
        scratch_shapes=[
            pltpu.VMEM((_S, _S), jnp.float32),
            pltpu.VMEM((1, _S), jnp.float32),
            pltpu.SMEM((1, 1), jnp.float32),
        ],
    )(z_flat, codebook, adjacency, p_col)

    latent = z_real.shape[-1]
    z_complex = lax.complex(zq[:, :latent], zq[:, latent:])
    return z_complex, loss[0, 0], idx.reshape(_N), pp[0, 0]
